# baseline (speedup 1.0000x reference)
import functools

import jax
import jax.numpy as jnp
from jax import lax
from jax.experimental import pallas as pl
from jax.experimental.pallas import tpu as pltpu
from jax.experimental.pallas import tpu_sc as plsc

NUM_EXPERTS = 8
INPUT_DIM = 768
TOKEN_BLOCK = 4096


def _body(x_ref, wt_ref, b_ref, out_ref):
    logits = (
        jnp.dot(x_ref[...], wt_ref[...], preferred_element_type=jnp.float32)
        + b_ref[...]
    )
    out_ref[...] = logits.T


def kernel(x, W, b):
    bsz, seq, dim = x.shape
    n_tokens = bsz * seq
    xf = x.reshape(n_tokens, dim)
    out = pl.pallas_call(
        _body,
        grid=(n_tokens // TOKEN_BLOCK,),
        in_specs=[
            pl.BlockSpec((TOKEN_BLOCK, INPUT_DIM), lambda i: (i, 0)),
            pl.BlockSpec((INPUT_DIM, NUM_EXPERTS), lambda i: (0, 0)),
            pl.BlockSpec((1, NUM_EXPERTS), lambda i: (0, 0)),
        ],
        out_specs=pl.BlockSpec((NUM_EXPERTS, TOKEN_BLOCK), lambda i: (0, i)),
        out_shape=jax.ShapeDtypeStruct((NUM_EXPERTS, n_tokens), jnp.float32),
    )(xf, W.T, b.reshape(1, NUM_EXPERTS))

    tpw = n_tokens // 32
    mesh = plsc.VectorSubcoreMesh(
        core_axis_name="c", subcore_axis_name="s", num_cores=2, num_subcores=16
    )

    def _copy_body(logits_hbm, out_hbm, lg_v):
        wid = lax.axis_index("s") * 2 + lax.axis_index("c")
        base = wid * tpw
        pltpu.sync_copy(logits_hbm.at[:, pl.ds(base, tpw)], lg_v)
        pltpu.sync_copy(lg_v, out_hbm.at[:, pl.ds(base, tpw)])

    f = pl.kernel(
        _copy_body,
        out_type=jax.ShapeDtypeStruct((NUM_EXPERTS, n_tokens), jnp.float32),
        mesh=mesh,
        scratch_types=[pltpu.VMEM((NUM_EXPERTS, tpw), jnp.float32)],
        compiler_params=pltpu.CompilerParams(needs_layout_passes=False),
    )
    return f(out)


# speedup vs baseline: 1.4349x; 1.4349x over previous
import jax
import jax.numpy as jnp
from jax.experimental import pallas as pl

NUM_EXPERTS = 8
INPUT_DIM = 768
TOKEN_BLOCK = 8192


def _body(x_ref, wt_ref, b_ref, out_ref):
    logits = (
        jnp.dot(x_ref[...], wt_ref[...], preferred_element_type=jnp.float32)
        + b_ref[...]
    )
    out_ref[...] = logits.T


def kernel(x, W, b):
    bsz, seq, dim = x.shape
    n_tokens = bsz * seq
    xf = x.reshape(n_tokens, dim)
    out = pl.pallas_call(
        _body,
        grid=(n_tokens // TOKEN_BLOCK,),
        in_specs=[
            pl.BlockSpec((TOKEN_BLOCK, INPUT_DIM), lambda i: (i, 0)),
            pl.BlockSpec((INPUT_DIM, NUM_EXPERTS), lambda i: (0, 0)),
            pl.BlockSpec((1, NUM_EXPERTS), lambda i: (0, 0)),
        ],
        out_specs=pl.BlockSpec((NUM_EXPERTS, TOKEN_BLOCK), lambda i: (0, i)),
        out_shape=jax.ShapeDtypeStruct((NUM_EXPERTS, n_tokens), jnp.float32),
    )(xf, W.T, b.reshape(1, NUM_EXPERTS))
    return out


# speedup vs baseline: 1.4925x; 1.0402x over previous
import jax
import jax.numpy as jnp
from jax.experimental import pallas as pl

NUM_EXPERTS = 8
INPUT_DIM = 768
TOKEN_BLOCK = 2048


def _body(x_ref, wt_ref, b_ref, out_ref):
    logits = (
        jnp.dot(x_ref[...], wt_ref[...], preferred_element_type=jnp.float32)
        + b_ref[...]
    )
    out_ref[...] = logits.T


def kernel(x, W, b):
    bsz, seq, dim = x.shape
    n_tokens = bsz * seq
    xf = x.reshape(n_tokens, dim)
    out = pl.pallas_call(
        _body,
        grid=(n_tokens // TOKEN_BLOCK,),
        in_specs=[
            pl.BlockSpec((TOKEN_BLOCK, INPUT_DIM), lambda i: (i, 0)),
            pl.BlockSpec((INPUT_DIM, NUM_EXPERTS), lambda i: (0, 0)),
            pl.BlockSpec((1, NUM_EXPERTS), lambda i: (0, 0)),
        ],
        out_specs=pl.BlockSpec((NUM_EXPERTS, TOKEN_BLOCK), lambda i: (0, i)),
        out_shape=jax.ShapeDtypeStruct((NUM_EXPERTS, n_tokens), jnp.float32),
    )(xf, W.T, b.reshape(1, NUM_EXPERTS))
    return out


# speedup vs baseline: 1.5120x; 1.0131x over previous
import jax
import jax.numpy as jnp
from jax import lax
from jax.experimental import pallas as pl

NUM_EXPERTS = 8
INPUT_DIM = 768
TOKEN_BLOCK = 4096


def _body(x_ref, wt_ref, b_ref, out_ref):
    logits = (
        jnp.dot(x_ref[...], wt_ref[...], preferred_element_type=jnp.float32)
        + b_ref[...]
    ).T
    e = lax.broadcasted_iota(jnp.int32, logits.shape, 0)
    m1 = jnp.max(logits, axis=0, keepdims=True)
    i1 = jnp.min(jnp.where(logits == m1, e, NUM_EXPERTS), axis=0, keepdims=True)
    c1 = e == i1
    lp = jnp.where(c1, -jnp.inf, logits)
    m2 = jnp.max(lp, axis=0, keepdims=True)
    i2 = jnp.min(jnp.where(lp == m2, e, NUM_EXPERTS), axis=0, keepdims=True)
    w1 = 1.0 / (1.0 + jnp.exp(m2 - m1))
    out_ref[...] = jnp.where(c1, w1, jnp.where(e == i2, 1.0 - w1, 0.0))


def kernel(x, W, b):
    bsz, seq, dim = x.shape
    n_tokens = bsz * seq
    xf = x.reshape(n_tokens, dim)
    out = pl.pallas_call(
        _body,
        grid=(n_tokens // TOKEN_BLOCK,),
        in_specs=[
            pl.BlockSpec((TOKEN_BLOCK, INPUT_DIM), lambda i: (i, 0)),
            pl.BlockSpec((INPUT_DIM, NUM_EXPERTS), lambda i: (0, 0)),
            pl.BlockSpec((1, NUM_EXPERTS), lambda i: (0, 0)),
        ],
        out_specs=pl.BlockSpec((NUM_EXPERTS, TOKEN_BLOCK), lambda i: (0, i)),
        out_shape=jax.ShapeDtypeStruct((NUM_EXPERTS, n_tokens), jnp.float32),
    )(xf, W.T, b.reshape(1, NUM_EXPERTS))
    return out.T.reshape(bsz, seq, NUM_EXPERTS)
